# baseline (speedup 1.0000x reference)
import functools

import jax
import jax.numpy as jnp
from jax import lax
from jax.experimental import pallas as pl
from jax.experimental.pallas import tpu as pltpu
from jax.experimental.pallas import tpu_sc as plsc

_V = 100000
_NF = 26
_D = 32
_EPS = 1e-5
_L = 16
_FEAT = _NF * _D


_TAKE_DNUMS = lax.GatherDimensionNumbers(
    offset_dims=(), collapsed_slice_dims=(0,), start_index_map=(0,))


def _permute(v, idx):
    return lax.gather(v, idx[:, None], _TAKE_DNUMS, (1,),
                      mode=lax.GatherScatterMode.PROMISE_IN_BOUNDS)


def _lanesum(v):
    for k in (1, 2, 4, 8):
        idx = jnp.arange(_L, dtype=jnp.int32) ^ k
        v = v + _permute(v, idx)
    return v


def _rsqrt(x):
    i = lax.bitcast_convert_type(x, jnp.int32)
    i = jnp.int32(0x5F3759DF) - lax.shift_right_logical(i, 1)
    y = lax.bitcast_convert_type(i, jnp.float32)
    for _ in range(4):
        y = y * (1.5 - 0.5 * x * y * y)
    return y


def kernel(x, tables, gamma, beta):
    B = x.shape[0]
    info = plsc.get_sparse_core_info()
    NC, NS = info.num_cores, info.num_subcores
    NW = NC * NS
    RW = B // NW
    CH = 16
    NCH = RW // CH
    IC = CH * _NF
    G = 104
    NG = IC // G

    offs = (jnp.arange(_NF, dtype=jnp.int32) * (_V + 1))[None, :]
    idx_flat = (jnp.clip(x, 0, _V) + offs).reshape(-1)
    tab2 = tables.reshape(_NF * (_V + 1), _D)

    mesh = plsc.VectorSubcoreMesh(core_axis_name="c", subcore_axis_name="s")

    @functools.partial(
        pl.kernel,
        mesh=mesh,
        compiler_params=pltpu.CompilerParams(use_tc_tiling_on_sc=False),
        out_type=jax.ShapeDtypeStruct((B * _NF, _D), jnp.float32),
        scratch_types=[
            pltpu.VMEM((RW * _NF,), jnp.int32),
            pltpu.VMEM((IC, _D), jnp.float32),
            pltpu.VMEM((_FEAT,), jnp.float32),
            pltpu.VMEM((_FEAT,), jnp.float32),
            pltpu.SemaphoreType.DMA,
        ],
    )
    def run(idx_hbm, tab_hbm, gam_hbm, bet_hbm, out_hbm,
            idx_v, emb_v, gam_v, bet_v, gsem):
        wid = lax.axis_index("s") * NC + lax.axis_index("c")
        ibase = wid * (RW * _NF)
        pltpu.sync_copy(idx_hbm.at[pl.ds(ibase, RW * _NF)], idx_v)
        pltpu.sync_copy(gam_hbm, gam_v)
        pltpu.sync_copy(bet_hbm, bet_v)

        def chunk_body(ch, carry):
            cbase = pl.multiple_of(ch * IC, 8)
            copies = []
            for g in range(NG):
                off = pl.multiple_of(cbase + g * G, 8)
                copies.append(pltpu.async_copy(
                    tab_hbm.at[idx_v.at[pl.ds(off, G)]],
                    emb_v.at[pl.ds(g * G, G)],
                    gsem,
                ))
            for c in copies:
                c.wait()

            def row_body(r, c2):
                rb = r * _NF
                s = jnp.zeros((_L,), jnp.float32)
                s2 = jnp.zeros((_L,), jnp.float32)
                for t in range(_NF):
                    for h in range(2):
                        v = emb_v[rb + t, pl.ds(h * _L, _L)]
                        s = s + v
                        s2 = s2 + v * v
                tot = _lanesum(s)
                tot2 = _lanesum(s2)
                mu = tot * (1.0 / _FEAT)
                var = tot2 * (1.0 / _FEAT) - mu * mu
                rstd = _rsqrt(var + _EPS)
                for t in range(_NF):
                    for h in range(2):
                        fo = t * _D + h * _L
                        v = emb_v[rb + t, pl.ds(h * _L, _L)]
                        gm = gam_v[pl.ds(fo, _L)]
                        bt = bet_v[pl.ds(fo, _L)]
                        emb_v[rb + t, pl.ds(h * _L, _L)] = (v - mu) * rstd * gm + bt
                return c2

            lax.fori_loop(0, CH, row_body, 0)
            pltpu.sync_copy(emb_v, out_hbm.at[pl.ds(ibase + cbase, IC)])
            return carry

        lax.fori_loop(0, NCH, chunk_body, 0)

    out = run(idx_flat, tab2, gamma, beta)
    return out.reshape(B, _FEAT)


# speedup vs baseline: 2.0520x; 2.0520x over previous
import functools

import jax
import jax.numpy as jnp
from jax import lax
from jax.experimental import pallas as pl
from jax.experimental.pallas import tpu as pltpu
from jax.experimental.pallas import tpu_sc as plsc

_V = 100000
_NF = 26
_D = 32
_EPS = 1e-5
_L = 16
_FEAT = _NF * _D
_VP = _V + 4


_TAKE_DNUMS = lax.GatherDimensionNumbers(
    offset_dims=(), collapsed_slice_dims=(0,), start_index_map=(0,))


def _permute(v, idx):
    return lax.gather(v, idx[:, None], _TAKE_DNUMS, (1,),
                      mode=lax.GatherScatterMode.PROMISE_IN_BOUNDS)


def _lanesum(v):
    for k in (1, 2, 4, 8):
        idx = jnp.arange(_L, dtype=jnp.int32) ^ k
        v = v + _permute(v, idx)
    return v


def _rsqrt(x):
    i = lax.bitcast_convert_type(x, jnp.int32)
    i = jnp.int32(0x5F3759DF) - lax.shift_right_logical(i, 1)
    y = lax.bitcast_convert_type(i, jnp.float32)
    for _ in range(4):
        y = y * (1.5 - 0.5 * x * y * y)
    return y


def kernel(x, tables, gamma, beta):
    B = x.shape[0]
    info = plsc.get_sparse_core_info()
    NC, NS = info.num_cores, info.num_subcores
    NW = NC * NS
    RW = B // NW
    CH = 8
    NCH = RW // CH
    IC = CH * _NF
    G = 104
    NG = IC // G
    ORC = CH * _FEAT // 128

    offs = (jnp.arange(_NF, dtype=jnp.int32) * _VP)[None, :]
    idx = jnp.clip(x, 0, _V) + offs
    gidx = (idx >> 2).reshape(-1)
    qoff = jnp.pad((idx & 3) * _D, ((0, 0), (0, 32 - _NF))).reshape(-1)
    tabp = jnp.pad(tables, ((0, 0), (0, 3), (0, 0))).reshape(-1, 128)

    mesh = plsc.VectorSubcoreMesh(core_axis_name="c", subcore_axis_name="s")

    @functools.partial(
        pl.kernel,
        mesh=mesh,
        compiler_params=pltpu.CompilerParams(use_tc_tiling_on_sc=False),
        out_type=jax.ShapeDtypeStruct((B * _FEAT // 128, 128), jnp.float32),
        scratch_types=[
            pltpu.VMEM((RW * _NF,), jnp.int32),
            pltpu.VMEM((RW * 32,), jnp.int32),
            pltpu.VMEM((IC, 128), jnp.float32),
            pltpu.VMEM((ORC, 128), jnp.float32),
            pltpu.VMEM((_FEAT,), jnp.float32),
            pltpu.VMEM((_FEAT,), jnp.float32),
            pltpu.SemaphoreType.DMA,
        ],
    )
    def run(gidx_hbm, qoff_hbm, tab_hbm, gam_hbm, bet_hbm, out_hbm,
            gidx_v, qoff_v, emb_v, outb_v, gam_v, bet_v, gsem):
        wid = lax.axis_index("s") * NC + lax.axis_index("c")
        ibase = wid * (RW * _NF)
        obase = wid * (RW * _FEAT // 128)
        pltpu.sync_copy(gidx_hbm.at[pl.ds(ibase, RW * _NF)], gidx_v)
        pltpu.sync_copy(qoff_hbm.at[pl.ds(wid * (RW * 32), RW * 32)], qoff_v)
        pltpu.sync_copy(gam_hbm, gam_v)
        pltpu.sync_copy(bet_hbm, bet_v)

        def chunk_body(ch, carry):
            cbase = pl.multiple_of(ch * IC, 8)
            copies = []
            for g in range(NG):
                off = pl.multiple_of(cbase + g * G, 8)
                copies.append(pltpu.async_copy(
                    tab_hbm.at[gidx_v.at[pl.ds(off, G)]],
                    emb_v.at[pl.ds(g * G, G)],
                    gsem,
                ))
            for c in copies:
                c.wait()

            def row_body(r, c2):
                rb = r * _NF
                qrb = (ch * CH + r) * 32
                qv0 = qoff_v[pl.ds(qrb, _L)]
                qv1 = qoff_v[pl.ds(qrb + _L, _L)]
                s = jnp.zeros((_L,), jnp.float32)
                s2 = jnp.zeros((_L,), jnp.float32)
                qs = []
                for t in range(_NF):
                    qo = qv0[t] if t < _L else qv1[t - _L]
                    qs.append(qo)
                    for h in range(2):
                        v = emb_v[rb + t, pl.ds(qo + h * _L, _L)]
                        s = s + v
                        s2 = s2 + v * v
                tot = _lanesum(s)
                tot2 = _lanesum(s2)
                mu = tot * (1.0 / _FEAT)
                var = tot2 * (1.0 / _FEAT) - mu * mu
                rstd = _rsqrt(var + _EPS)
                fl0 = r * _FEAT
                for t in range(_NF):
                    qo = qs[t]
                    for h in range(2):
                        fo = t * _D + h * _L
                        v = emb_v[rb + t, pl.ds(qo + h * _L, _L)]
                        gm = gam_v[pl.ds(fo, _L)]
                        bt = bet_v[pl.ds(fo, _L)]
                        y = (v - mu) * rstd * gm + bt
                        fl = fl0 + fo
                        outb_v[lax.shift_right_logical(fl, 7),
                               pl.ds(lax.bitwise_and(fl, 127), _L)] = y
                return c2

            lax.fori_loop(0, CH, row_body, 0)
            pltpu.sync_copy(outb_v,
                            out_hbm.at[pl.ds(obase + ch * ORC, ORC)])
            return carry

        lax.fori_loop(0, NCH, chunk_body, 0)

    out = run(gidx, qoff, tabp, gamma, beta)
    return out.reshape(B, _FEAT)


# speedup vs baseline: 2.2720x; 1.1072x over previous
import functools

import jax
import jax.numpy as jnp
from jax import lax
from jax.experimental import pallas as pl
from jax.experimental.pallas import tpu as pltpu
from jax.experimental.pallas import tpu_sc as plsc

_V = 100000
_NF = 26
_D = 32
_EPS = 1e-5
_L = 16
_FEAT = _NF * _D
_VP = _V + 4


_TAKE_DNUMS = lax.GatherDimensionNumbers(
    offset_dims=(), collapsed_slice_dims=(0,), start_index_map=(0,))


def _permute(v, idx):
    return lax.gather(v, idx[:, None], _TAKE_DNUMS, (1,),
                      mode=lax.GatherScatterMode.PROMISE_IN_BOUNDS)


def _lanesum(v):
    for k in (1, 2, 4, 8):
        idx = jnp.arange(_L, dtype=jnp.int32) ^ k
        v = v + _permute(v, idx)
    return v


def _rsqrt(x):
    i = lax.bitcast_convert_type(x, jnp.int32)
    i = jnp.int32(0x5F3759DF) - lax.shift_right_logical(i, 1)
    y = lax.bitcast_convert_type(i, jnp.float32)
    for _ in range(4):
        y = y * (1.5 - 0.5 * x * y * y)
    return y


def kernel(x, tables, gamma, beta):
    B = x.shape[0]
    info = plsc.get_sparse_core_info()
    NC, NS = info.num_cores, info.num_subcores
    NW = NC * NS
    RW = B // NW
    CH = 8
    NCH = RW // CH
    IC = CH * _NF
    G = 104
    NG = IC // G
    ORC = CH * _FEAT // 128

    offs = (jnp.arange(_NF, dtype=jnp.int32) * (_V + 1))[None, :]
    idx = jnp.clip(x, 0, _V) + offs
    gidx = (idx >> 2).reshape(-1)
    qoff = jnp.pad((idx & 3) * _D, ((0, 0), (0, 32 - _NF))).reshape(-1)
    tabp = jnp.concatenate(
        [tables.reshape(-1), jnp.zeros((64,), jnp.float32)]).reshape(-1, 128)

    mesh = plsc.VectorSubcoreMesh(core_axis_name="c", subcore_axis_name="s")

    @functools.partial(
        pl.kernel,
        mesh=mesh,
        compiler_params=pltpu.CompilerParams(use_tc_tiling_on_sc=False),
        out_type=jax.ShapeDtypeStruct((B * _FEAT // 128, 128), jnp.float32),
        scratch_types=[
            pltpu.VMEM((RW * _NF,), jnp.int32),
            pltpu.VMEM((RW * 32,), jnp.int32),
            pltpu.VMEM((IC, 128), jnp.float32),
            pltpu.VMEM((ORC, 128), jnp.float32),
            pltpu.VMEM((_FEAT,), jnp.float32),
            pltpu.VMEM((_FEAT,), jnp.float32),
            pltpu.SemaphoreType.DMA,
        ],
    )
    def run(gidx_hbm, qoff_hbm, tab_hbm, gam_hbm, bet_hbm, out_hbm,
            gidx_v, qoff_v, emb_v, outb_v, gam_v, bet_v, gsem):
        wid = lax.axis_index("s") * NC + lax.axis_index("c")
        ibase = wid * (RW * _NF)
        obase = wid * (RW * _FEAT // 128)
        pltpu.sync_copy(gidx_hbm.at[pl.ds(ibase, RW * _NF)], gidx_v)
        pltpu.sync_copy(qoff_hbm.at[pl.ds(wid * (RW * 32), RW * 32)], qoff_v)
        pltpu.sync_copy(gam_hbm, gam_v)
        pltpu.sync_copy(bet_hbm, bet_v)

        def chunk_body(ch, carry):
            cbase = pl.multiple_of(ch * IC, 8)
            copies = []
            for g in range(NG):
                off = pl.multiple_of(cbase + g * G, 8)
                copies.append(pltpu.async_copy(
                    tab_hbm.at[gidx_v.at[pl.ds(off, G)]],
                    emb_v.at[pl.ds(g * G, G)],
                    gsem,
                ))
            for c in copies:
                c.wait()

            def row_body(r, c2):
                rb = r * _NF
                qrb = (ch * CH + r) * 32
                qv0 = qoff_v[pl.ds(qrb, _L)]
                qv1 = qoff_v[pl.ds(qrb + _L, _L)]
                s = jnp.zeros((_L,), jnp.float32)
                s2 = jnp.zeros((_L,), jnp.float32)
                qs = []
                for t in range(_NF):
                    qo = qv0[t] if t < _L else qv1[t - _L]
                    qs.append(qo)
                    for h in range(2):
                        v = emb_v[rb + t, pl.ds(qo + h * _L, _L)]
                        s = s + v
                        s2 = s2 + v * v
                tot = _lanesum(s)
                tot2 = _lanesum(s2)
                mu = tot * (1.0 / _FEAT)
                var = tot2 * (1.0 / _FEAT) - mu * mu
                rstd = _rsqrt(var + _EPS)
                fl0 = r * _FEAT
                for t in range(_NF):
                    qo = qs[t]
                    for h in range(2):
                        fo = t * _D + h * _L
                        v = emb_v[rb + t, pl.ds(qo + h * _L, _L)]
                        gm = gam_v[pl.ds(fo, _L)]
                        bt = bet_v[pl.ds(fo, _L)]
                        y = (v - mu) * rstd * gm + bt
                        fl = fl0 + fo
                        outb_v[lax.shift_right_logical(fl, 7),
                               pl.ds(lax.bitwise_and(fl, 127), _L)] = y
                return c2

            lax.fori_loop(0, CH, row_body, 0)
            pltpu.sync_copy(outb_v,
                            out_hbm.at[pl.ds(obase + ch * ORC, ORC)])
            return carry

        lax.fori_loop(0, NCH, chunk_body, 0)

    out = run(gidx, qoff, tabp, gamma, beta)
    return out.reshape(B, _FEAT)


# speedup vs baseline: 6.7169x; 2.9564x over previous
import jax
import jax.numpy as jnp
from jax import lax
from jax.experimental import pallas as pl
from jax.experimental.pallas import tpu as pltpu
from jax.experimental.pallas import tpu_sc as plsc

_V = 100000
_NF = 26
_D = 32
_EPS = 1e-5
_L = 16
_FEAT = _NF * _D
_VP = 100032
_GROWS = _NF * _VP // 4
_RPB = 16672
_RPG = _RPB // 4
_NBLK = _VP // _RPB


def _repack(tables):
    def body(t_ref, o_ref):
        t3 = t_ref[0].reshape(_RPG, 4, _D)
        for k in range(4):
            o_ref[:, k * _D:(k + 1) * _D] = t3[:, k, :]

    return pl.pallas_call(
        body,
        grid=(_NF, _NBLK),
        in_specs=[pl.BlockSpec((1, _RPB, _D), lambda f, c: (f, c, 0))],
        out_specs=pl.BlockSpec((_RPG, 128), lambda f, c: (f * _NBLK + c, 0)),
        out_shape=jax.ShapeDtypeStruct((_GROWS, 128), jnp.float32),
    )(tables)


def _layernorm(og, gamma, beta, B):
    RB = 128
    XR = RB * _FEAT // 128

    def body(g_ref, b_ref, x_ref, o_ref):
        x3 = x_ref[...].reshape(RB // 2, 13, 128)

        def rows(c, lo, hi):
            return x3[:, c, lo:hi]

        h_even = jnp.concatenate(
            [rows(c, 0, 128) for c in range(6)] + [rows(6, 0, 64)], axis=1)
        h_odd = jnp.concatenate(
            [rows(6, 64, 128)] + [rows(7 + c, 0, 128) for c in range(6)],
            axis=1)
        h = jnp.stack([h_even, h_odd], axis=1).reshape(RB, _FEAT)
        mu = jnp.mean(h, axis=1, keepdims=True)
        var = jnp.mean(h * h, axis=1, keepdims=True) - mu * mu
        hn = (h - mu) * lax.rsqrt(var + _EPS)
        o_ref[...] = hn * g_ref[...][None, :] + b_ref[...][None, :]

    return pl.pallas_call(
        body,
        grid=(B // RB,),
        in_specs=[
            pl.BlockSpec((_FEAT,), lambda r: (0,)),
            pl.BlockSpec((_FEAT,), lambda r: (0,)),
            pl.BlockSpec((XR, 128), lambda r: (r, 0)),
        ],
        out_specs=pl.BlockSpec((RB, _FEAT), lambda r: (r, 0)),
        out_shape=jax.ShapeDtypeStruct((B, _FEAT), jnp.float32),
    )(gamma, beta, og)


def kernel(x, tables, gamma, beta):
    B = x.shape[0]
    info = plsc.get_sparse_core_info()
    NC, NS = info.num_cores, info.num_subcores
    NW = NC * NS
    LPW = B * _NF // NW
    CL = 128
    NCH = LPW // CL
    ORC = CL * _D // 128
    IR = B * _NF // 128
    IRW = IR // NW

    offs = (jnp.arange(_NF, dtype=jnp.int32) * _VP)[None, :]
    idxp = (jnp.clip(x, 0, _V) + offs).reshape(IR, 128)
    tabp = _repack(tables)

    mesh = plsc.VectorSubcoreMesh(core_axis_name="c", subcore_axis_name="s")

    @pl.kernel(
        mesh=mesh,
        compiler_params=pltpu.CompilerParams(use_tc_tiling_on_sc=False),
        out_type=jax.ShapeDtypeStruct((B * _FEAT // 128, 128), jnp.float32),
        scratch_types=[
            pltpu.VMEM((IRW, 128), jnp.int32),
            pltpu.VMEM((CL,), jnp.int32),
            pltpu.VMEM((CL, 128), jnp.float32),
            pltpu.VMEM((ORC, 128), jnp.float32),
            pltpu.SemaphoreType.DMA,
        ],
    )
    def gather(idx_hbm, tab_hbm, out_hbm, idx_v, gsc_v, emb_v, outb_v, gsem):
        wid = lax.axis_index("s") * NC + lax.axis_index("c")
        pltpu.sync_copy(idx_hbm.at[pl.ds(wid * IRW, IRW)], idx_v)

        def chunk_body(c, carry):
            ivs = []
            for k in range(CL // _L):
                iv = idx_v[c, pl.ds(k * _L, _L)]
                gsc_v[pl.ds(k * _L, _L)] = lax.shift_right_logical(iv, 2)
                ivs.append(lax.bitwise_and(iv, 3) * _D)
            pltpu.async_copy(tab_hbm.at[gsc_v], emb_v, gsem).wait()
            for k in range(CL // _L):
                qv = ivs[k]
                for j in range(_L):
                    lk = k * _L + j
                    qo = qv[j]
                    for h in range(2):
                        outb_v[lk >> 2, pl.ds((lk & 3) * _D + h * _L, _L)] = (
                            emb_v[lk, pl.ds(qo + h * _L, _L)])
            pltpu.sync_copy(outb_v,
                            out_hbm.at[pl.ds(wid * (LPW * _D // 128) + c * ORC,
                                             ORC)])
            return carry

        lax.fori_loop(0, NCH, chunk_body, 0)

    og = gather(idxp, tabp)
    return _layernorm(og, gamma, beta, B)


# speedup vs baseline: 8.4663x; 1.2605x over previous
import jax
import jax.numpy as jnp
from jax import lax
from jax.experimental import pallas as pl
from jax.experimental.pallas import tpu as pltpu
from jax.experimental.pallas import tpu_sc as plsc

_V = 100000
_NF = 26
_D = 32
_EPS = 1e-5
_L = 16
_FEAT = _NF * _D
_VP = 100008
_GROWS = _NF * _VP // 4
_RPB = 16672
_RPG = _RPB // 4
_NBLK = _VP // _RPB


def _repack(tables):
    def body(t_ref, o_ref):
        t3 = t_ref[0].reshape(_RPG, 4, _D)
        for k in range(4):
            o_ref[:, k * _D:(k + 1) * _D] = t3[:, k, :]

    return pl.pallas_call(
        body,
        grid=(_NF, _NBLK),
        in_specs=[pl.BlockSpec((1, _RPB, _D), lambda f, c: (f, c, 0))],
        out_specs=pl.BlockSpec((_RPG, 128), lambda f, c: (f * _NBLK + c, 0)),
        out_shape=jax.ShapeDtypeStruct((_GROWS, 128), jnp.float32),
    )(tables)


def _layernorm(og, gamma, beta, B):
    RB = 128
    XR = RB * _FEAT // 128

    def body(g_ref, b_ref, x_ref, o_ref):
        x3 = x_ref[...].reshape(RB // 2, 13, 128)

        def rows(c, lo, hi):
            return x3[:, c, lo:hi]

        h_even = jnp.concatenate(
            [rows(c, 0, 128) for c in range(6)] + [rows(6, 0, 64)], axis=1)
        h_odd = jnp.concatenate(
            [rows(6, 64, 128)] + [rows(7 + c, 0, 128) for c in range(6)],
            axis=1)
        h = jnp.stack([h_even, h_odd], axis=1).reshape(RB, _FEAT)
        mu = jnp.mean(h, axis=1, keepdims=True)
        var = jnp.mean(h * h, axis=1, keepdims=True) - mu * mu
        hn = (h - mu) * lax.rsqrt(var + _EPS)
        o_ref[...] = hn * g_ref[...][None, :] + b_ref[...][None, :]

    return pl.pallas_call(
        body,
        grid=(B // RB,),
        in_specs=[
            pl.BlockSpec((_FEAT,), lambda r: (0,)),
            pl.BlockSpec((_FEAT,), lambda r: (0,)),
            pl.BlockSpec((XR, 128), lambda r: (r, 0)),
        ],
        out_specs=pl.BlockSpec((RB, _FEAT), lambda r: (r, 0)),
        out_shape=jax.ShapeDtypeStruct((B, _FEAT), jnp.float32),
    )(gamma, beta, og)


def kernel(x, tables, gamma, beta):
    B = x.shape[0]
    info = plsc.get_sparse_core_info()
    NC, NS = info.num_cores, info.num_subcores
    NW = NC * NS
    LPW = B * _NF // NW
    CL = 128
    NCH = LPW // CL
    ORC = CL * _D // 128
    IR = B * _NF // 128
    IRW = IR // NW

    offs = (jnp.arange(_NF, dtype=jnp.int32) * _VP)[None, :]
    idxp = (jnp.clip(x, 0, _V) + offs).reshape(IR, 128)
    tabp = jnp.pad(tables, ((0, 0), (0, _VP - _V - 1), (0, 96))).reshape(-1, 128)

    mesh = plsc.VectorSubcoreMesh(core_axis_name="c", subcore_axis_name="s")

    @pl.kernel(
        mesh=mesh,
        compiler_params=pltpu.CompilerParams(use_tc_tiling_on_sc=False),
        out_type=jax.ShapeDtypeStruct((B * _FEAT // 128, 128), jnp.float32),
        scratch_types=[
            pltpu.VMEM((IRW, 128), jnp.int32),
            pltpu.VMEM((CL,), jnp.int32),
            pltpu.VMEM((CL, 128), jnp.float32),
            pltpu.VMEM((ORC, 128), jnp.float32),
            pltpu.SemaphoreType.DMA,
        ],
    )
    def gather(idx_hbm, tab_hbm, out_hbm, idx_v, gsc_v, emb_v, outb_v, gsem):
        wid = lax.axis_index("s") * NC + lax.axis_index("c")
        pltpu.sync_copy(idx_hbm.at[pl.ds(wid * IRW, IRW)], idx_v)

        def chunk_body(c, carry):
            for k in range(CL // _L):
                gsc_v[pl.ds(k * _L, _L)] = idx_v[c, pl.ds(k * _L, _L)]
            pltpu.async_copy(tab_hbm.at[gsc_v], emb_v, gsem).wait()
            for lk in range(CL):
                for h in range(2):
                    outb_v[lk >> 2, pl.ds((lk & 3) * _D + h * _L, _L)] = (
                        emb_v[lk, pl.ds(h * _L, _L)])
            pltpu.sync_copy(outb_v,
                            out_hbm.at[pl.ds(wid * (LPW * _D // 128) + c * ORC,
                                             ORC)])
            return carry

        lax.fori_loop(0, NCH, chunk_body, 0)

    og = gather(idxp, tabp)
    return _layernorm(og, gamma, beta, B)
